# baseline (speedup 1.0000x reference)
import functools

import jax
import jax.numpy as jnp
from jax import lax
from jax.experimental import pallas as pl
from jax.experimental.pallas import tpu as pltpu
from jax.experimental.pallas import tpu_sc as plsc

_VB = 2048


def _sc_gather(idx, table):
    (B,) = idx.shape
    V, D = table.shape
    info = plsc.get_sparse_core_info()
    NC, NS = info.num_cores, info.num_subcores
    NW = NC * NS
    b_per_w = B // NW
    mesh = plsc.VectorSubcoreMesh(core_axis_name="c", subcore_axis_name="s")

    @functools.partial(
        pl.kernel,
        mesh=mesh,
        out_type=jax.ShapeDtypeStruct((B, D), jnp.float32),
        scratch_types=[
            pltpu.VMEM((b_per_w,), jnp.int32),
            pltpu.VMEM((b_per_w, D), jnp.float32),
            pltpu.SemaphoreType.DMA,
        ],
    )
    def gather_kernel(idx_hbm, table_hbm, out_hbm, idx_v, rows_v, sem):
        wid = lax.axis_index("s") * NC + lax.axis_index("c")
        base = wid * b_per_w
        pltpu.sync_copy(idx_hbm.at[pl.ds(base, b_per_w)], idx_v)
        pltpu.async_copy(table_hbm.at[idx_v], rows_v, sem).wait()
        pltpu.sync_copy(rows_v, out_hbm.at[pl.ds(base, b_per_w)])

    return gather_kernel(idx, table)


def _add_body(tok_ref, pos_ref, out_ref):
    out_ref[...] = (tok_ref[...] + pos_ref[...]).astype(jnp.bfloat16)


def _add_cast(tok, pos):
    S, E = tok.shape
    return pl.pallas_call(
        _add_body,
        out_shape=jax.ShapeDtypeStruct((S, E), jnp.bfloat16),
    )(tok, pos)


def _mm_body(x_ref, w_ref, out_ref):
    out_ref[:, 0, :] = lax.dot_general(
        w_ref[...].astype(jnp.bfloat16),
        x_ref[...],
        (((1,), (1,)), ((), ())),
        preferred_element_type=jnp.float32,
    )


def _mm_t(x, W_out):
    S, E = x.shape
    V = W_out.shape[0]
    n_tiles = pl.cdiv(V, _VB)
    return pl.pallas_call(
        _mm_body,
        grid=(n_tiles,),
        in_specs=[
            pl.BlockSpec((S, E), lambda i: (0, 0)),
            pl.BlockSpec((_VB, E), lambda i: (i, 0)),
        ],
        out_specs=pl.BlockSpec((_VB, 1, S), lambda i: (i, 0, 0)),
        out_shape=jax.ShapeDtypeStruct((V, 1, S), jnp.float32),
    )(x, W_out)


def kernel(in_idx, tok_emb, pos_emb, W_out):
    B, S = in_idx.shape
    V, E = tok_emb.shape
    tok = _sc_gather(in_idx.reshape(-1), tok_emb)
    x = _add_cast(tok, pos_emb[:S])
    logits_t = _mm_t(x, W_out)
    return jnp.transpose(logits_t, (1, 2, 0))


# speedup vs baseline: 1.0357x; 1.0357x over previous
import functools

import jax
import jax.numpy as jnp
from jax import lax
from jax.experimental import pallas as pl
from jax.experimental.pallas import tpu as pltpu
from jax.experimental.pallas import tpu_sc as plsc

_VB = 1024


def _sc_gather(idx, table):
    (B,) = idx.shape
    V, D = table.shape
    info = plsc.get_sparse_core_info()
    NC, NS = info.num_cores, info.num_subcores
    NW = NC * NS
    b_per_w = B // NW
    mesh = plsc.VectorSubcoreMesh(core_axis_name="c", subcore_axis_name="s")

    @functools.partial(
        pl.kernel,
        mesh=mesh,
        out_type=jax.ShapeDtypeStruct((B, D), jnp.float32),
        scratch_types=[
            pltpu.VMEM((b_per_w,), jnp.int32),
            pltpu.VMEM((b_per_w, D), jnp.float32),
            pltpu.SemaphoreType.DMA,
        ],
    )
    def gather_kernel(idx_hbm, table_hbm, out_hbm, idx_v, rows_v, sem):
        wid = lax.axis_index("s") * NC + lax.axis_index("c")
        base = wid * b_per_w
        pltpu.sync_copy(idx_hbm.at[pl.ds(base, b_per_w)], idx_v)
        pltpu.async_copy(table_hbm.at[idx_v], rows_v, sem).wait()
        pltpu.sync_copy(rows_v, out_hbm.at[pl.ds(base, b_per_w)])

    return gather_kernel(idx, table)


def _add_body(tok_ref, pos_ref, out_ref):
    out_ref[...] = (tok_ref[...] + pos_ref[...]).astype(jnp.bfloat16)


def _add_cast(tok, pos):
    S, E = tok.shape
    return pl.pallas_call(
        _add_body,
        out_shape=jax.ShapeDtypeStruct((S, E), jnp.bfloat16),
    )(tok, pos)


def _mm_body(x_ref, w_ref, out_ref):
    out_ref[:, 0, :] = lax.dot_general(
        w_ref[...].astype(jnp.bfloat16),
        x_ref[...],
        (((1,), (1,)), ((), ())),
        preferred_element_type=jnp.float32,
    )


def _mm_t(x, W_out):
    S, E = x.shape
    V = W_out.shape[0]
    n_tiles = pl.cdiv(V, _VB)
    return pl.pallas_call(
        _mm_body,
        grid=(n_tiles,),
        in_specs=[
            pl.BlockSpec((S, E), lambda i: (0, 0)),
            pl.BlockSpec((_VB, E), lambda i: (i, 0)),
        ],
        out_specs=pl.BlockSpec((_VB, 1, S), lambda i: (i, 0, 0)),
        out_shape=jax.ShapeDtypeStruct((V, 1, S), jnp.float32),
    )(x, W_out)


def kernel(in_idx, tok_emb, pos_emb, W_out):
    B, S = in_idx.shape
    V, E = tok_emb.shape
    tok = _sc_gather(in_idx.reshape(-1), tok_emb)
    x = _add_cast(tok, pos_emb[:S])
    logits_t = _mm_t(x, W_out)
    return jnp.transpose(logits_t, (1, 2, 0))
